# Initial kernel scaffold; baseline (speedup 1.0000x reference)
import jax
import jax.numpy as jnp
from jax.experimental import pallas as pl


def kernel(conts, cates, combs, cate_table, comb_table):
    raise NotImplementedError("write your pallas kernel here")


# speedup vs baseline: 213.1027x; 213.1027x over previous
import functools

import jax
import jax.numpy as jnp
from jax import lax
from jax.experimental import pallas as pl
from jax.experimental.pallas import tpu as pltpu
from jax.experimental.pallas import tpu_sc as plsc

B = 16384
CONT_F = 13
CATE_F = 26
COMB_F = 325

NC = 2
NS = 16
NW = NC * NS
ROWS_PER_W = B // NW
CHUNK = 128
NCHUNK = ROWS_PER_W // CHUNK

_mesh = plsc.VectorSubcoreMesh(core_axis_name="c", subcore_axis_name="s")


@functools.partial(
    pl.kernel,
    mesh=_mesh,
    out_type=jax.ShapeDtypeStruct((B,), jnp.float32),
    scratch_types=[
        pltpu.VMEM((COMB_F * CHUNK,), jnp.int32),
        pltpu.VMEM((COMB_F * CHUNK,), jnp.float32),
        pltpu.VMEM((CATE_F * CHUNK,), jnp.int32),
        pltpu.VMEM((CATE_F * CHUNK,), jnp.float32),
        pltpu.VMEM((CONT_F * CHUNK,), jnp.float32),
        pltpu.VMEM((CONT_F * 16,), jnp.float32),
        pltpu.VMEM((ROWS_PER_W,), jnp.float32),
        pltpu.SemaphoreType.DMA,
    ],
)
def _poly2_sc(conts_r, cates_r, combs_r, cate_tab, comb_tab, wbc_hbm, out_hbm,
              comb_idx_v, comb_val_v, cate_idx_v, cate_val_v, cont_v,
              w_v, out_v, sem):
    wid = lax.axis_index("s") * NC + lax.axis_index("c")

    pltpu.sync_copy(wbc_hbm, w_v)

    for c in range(NCHUNK):
        blk = wid * NCHUNK + c
        pltpu.sync_copy(combs_r.at[pl.ds(blk * COMB_F * CHUNK, COMB_F * CHUNK)],
                        comb_idx_v)
        pltpu.sync_copy(cates_r.at[pl.ds(blk * CATE_F * CHUNK, CATE_F * CHUNK)],
                        cate_idx_v)
        pltpu.sync_copy(conts_r.at[pl.ds(blk * CONT_F * CHUNK, CONT_F * CHUNK)],
                        cont_v)

        cp_comb = pltpu.async_copy(comb_tab.at[comb_idx_v], comb_val_v, sem)
        cp_cate = pltpu.async_copy(cate_tab.at[cate_idx_v], cate_val_v, sem)
        cp_comb.wait()
        cp_cate.wait()

        for bs in range(CHUNK // 16):
            so = bs * 16

            def body_comb(f, acc):
                return acc + comb_val_v[pl.ds(f * CHUNK + so, 16)]

            acc = lax.fori_loop(0, COMB_F, body_comb,
                                jnp.zeros((16,), jnp.float32))

            def body_cate(f, acc):
                return acc + cate_val_v[pl.ds(f * CHUNK + so, 16)]

            acc = lax.fori_loop(0, CATE_F, body_cate, acc)

            def body_cont(f, acc):
                return acc + (cont_v[pl.ds(f * CHUNK + so, 16)]
                              * w_v[pl.ds(f * 16, 16)])

            acc = lax.fori_loop(0, CONT_F, body_cont, acc)

            out_v[pl.ds(c * CHUNK + so, 16)] = 1.0 / (1.0 + jnp.exp(-acc))

    pltpu.sync_copy(out_v, out_hbm.at[pl.ds(wid * ROWS_PER_W, ROWS_PER_W)])


def _field_major_blocks(x, nfields, dtype):
    x = x.reshape(NW, NCHUNK, CHUNK, nfields).transpose(0, 1, 3, 2)
    return x.reshape(-1).astype(dtype)


def kernel(conts, cates, combs, cate_table, comb_table):
    conts_r = _field_major_blocks(conts, CONT_F, jnp.float32)
    cates_r = _field_major_blocks(cates, CATE_F, jnp.int32)
    combs_r = _field_major_blocks(combs, COMB_F, jnp.int32)
    wbc = jnp.repeat(cate_table[:CONT_F, 0], 16)
    out = _poly2_sc(conts_r, cates_r, combs_r,
                    cate_table.reshape(-1), comb_table.reshape(-1), wbc)
    return out.reshape(B, 1)


# speedup vs baseline: 246.9359x; 1.1588x over previous
import functools

import jax
import jax.numpy as jnp
from jax import lax
from jax.experimental import pallas as pl
from jax.experimental.pallas import tpu as pltpu
from jax.experimental.pallas import tpu_sc as plsc

B = 16384
CONT_F = 13
CATE_F = 26
COMB_F = 325

NC = 2
NS = 16
NW = NC * NS
ROWS_PER_W = B // NW
CHUNK = 128
NCHUNK = ROWS_PER_W // CHUNK

COMB_N = COMB_F * CHUNK
CATE_N = CATE_F * CHUNK
CONT_N = CONT_F * CHUNK

_mesh = plsc.VectorSubcoreMesh(core_axis_name="c", subcore_axis_name="s")


@functools.partial(
    pl.kernel,
    mesh=_mesh,
    out_type=jax.ShapeDtypeStruct((B,), jnp.float32),
    scratch_types=[
        pltpu.VMEM((COMB_F, CHUNK), jnp.int32),
        pltpu.VMEM((COMB_F, CHUNK), jnp.float32),
        pltpu.VMEM((CATE_F, CHUNK), jnp.int32),
        pltpu.VMEM((CATE_F, CHUNK), jnp.float32),
        pltpu.VMEM((CONT_F, CHUNK), jnp.float32),
        pltpu.VMEM((CONT_F * 16,), jnp.float32),
        pltpu.VMEM((ROWS_PER_W,), jnp.float32),
        pltpu.SemaphoreType.DMA,
    ],
)
def _poly2_sc(conts_t, cates_t, combs_t, cate_tab, comb_tab, wbc_hbm,
              out_hbm,
              comb_idx_v, comb_val_v, cate_idx_v, cate_val_v, cont_v,
              w_v, out_v, sem):
    wid = lax.axis_index("s") * NC + lax.axis_index("c")
    base = wid * ROWS_PER_W

    pltpu.sync_copy(wbc_hbm, w_v)

    def chunk_body(c, carry):
        rb = base + c * CHUNK
        pltpu.sync_copy(combs_t.at[:, pl.ds(rb, CHUNK)], comb_idx_v)
        pltpu.sync_copy(cates_t.at[:, pl.ds(rb, CHUNK)], cate_idx_v)
        pltpu.sync_copy(conts_t.at[:, pl.ds(rb, CHUNK)], cont_v)

        def fire_comb(f, carry2):
            pltpu.async_copy(comb_tab.at[comb_idx_v.at[f]],
                             comb_val_v.at[f], sem)
            return carry2

        lax.fori_loop(0, COMB_F, fire_comb, jnp.int32(0))

        def fire_cate(f, carry2):
            pltpu.async_copy(cate_tab.at[cate_idx_v.at[f]],
                             cate_val_v.at[f], sem)
            return carry2

        lax.fori_loop(0, CATE_F, fire_cate, jnp.int32(0))

        def drain_comb(f, carry2):
            pltpu.make_async_copy(comb_tab.at[pl.ds(0, CHUNK)],
                                  comb_val_v.at[f], sem).wait()
            return carry2

        lax.fori_loop(0, COMB_F, drain_comb, jnp.int32(0))

        def drain_cate(f, carry2):
            pltpu.make_async_copy(cate_tab.at[pl.ds(0, CHUNK)],
                                  cate_val_v.at[f], sem).wait()
            return carry2

        lax.fori_loop(0, CATE_F, drain_cate, jnp.int32(0))

        def group_body(bs, carry2):
            so = bs * 16

            def body_comb(f, acc):
                return acc + comb_val_v[f, pl.ds(so, 16)]

            acc = lax.fori_loop(0, COMB_F, body_comb,
                                jnp.zeros((16,), jnp.float32))

            def body_cate(f, acc):
                return acc + cate_val_v[f, pl.ds(so, 16)]

            acc = lax.fori_loop(0, CATE_F, body_cate, acc)

            def body_cont(f, acc):
                return acc + (cont_v[f, pl.ds(so, 16)]
                              * w_v[pl.ds(f * 16, 16)])

            acc = lax.fori_loop(0, CONT_F, body_cont, acc)

            out_v[pl.ds(c * CHUNK + so, 16)] = 1.0 / (1.0 + jnp.exp(-acc))
            return carry2

        return lax.fori_loop(0, CHUNK // 16, group_body, carry)

    lax.fori_loop(0, NCHUNK, chunk_body, jnp.int32(0))

    pltpu.sync_copy(out_v, out_hbm.at[pl.ds(base, ROWS_PER_W)])


def kernel(conts, cates, combs, cate_table, comb_table):
    wbc = jnp.repeat(cate_table[:CONT_F, 0], 16)
    out = _poly2_sc(conts.T, cates.T.astype(jnp.int32),
                    combs.T.astype(jnp.int32),
                    cate_table.reshape(-1), comb_table.reshape(-1), wbc)
    return out.reshape(B, 1)
